# baseline (speedup 1.0000x reference)
import functools

import jax
import jax.numpy as jnp
from jax import lax
from jax.experimental import pallas as pl
from jax.experimental.pallas import tpu as pltpu
from jax.experimental.pallas import tpu_sc as plsc

L = 16
NC, NS = 2, 16
NW = NC * NS


@functools.cache
def _build(B, C, HW, S, interpret=False):
    RPW = (B * C) // NW
    CCH = min(128, RPW)
    PV = HW // L
    mesh = plsc.VectorSubcoreMesh(core_axis_name="c", subcore_axis_name="s")

    @functools.partial(
        pl.kernel,
        mesh=mesh,
        out_type=jax.ShapeDtypeStruct((B * C, HW), jnp.float32),
        scratch_types=[
            pltpu.VMEM((HW,), jnp.int32),
            pltpu.VMEM((CCH * S,), jnp.float32),
            pltpu.VMEM((CCH * S,), jnp.float32),
            pltpu.VMEM((HW,), jnp.float32),
            pltpu.VMEM((HW,), jnp.float32),
        ],
        compiler_params=pltpu.CompilerParams(needs_layout_passes=False),
        interpret=interpret,
    )
    def _slotfa(x_hbm, slot_hbm, a_hbm, b_hbm, out_hbm,
                slot_v, at_v, bt_v, x_v, y_v):
        wid = lax.axis_index("s") * NC + lax.axis_index("c")
        r0 = wid * RPW
        bidx = r0 // C
        c0 = r0 % C
        pltpu.sync_copy(slot_hbm.at[bidx], slot_v)
        for cc in range(RPW // CCH):
            cbase = c0 + cc * CCH
            rbase = r0 + cc * CCH
            pltpu.sync_copy(a_hbm.at[pl.ds(cbase * S, CCH * S)], at_v)
            pltpu.sync_copy(b_hbm.at[pl.ds(cbase * S, CCH * S)], bt_v)

            def row_body(ci, _, rbase=rbase):
                r = rbase + ci
                pltpu.sync_copy(x_hbm.at[r], x_v)
                cvec = jnp.full((L,), ci * S, jnp.int32)

                def pv_body(pv, _):
                    o = pv * L
                    idx = slot_v[pl.ds(o, L)] + cvec
                    av = plsc.load_gather(at_v, [idx])
                    bv = plsc.load_gather(bt_v, [idx])
                    y_v[pl.ds(o, L)] = av * x_v[pl.ds(o, L)] + bv
                    return 0

                lax.fori_loop(0, PV, pv_body, 0, unroll=8)
                pltpu.sync_copy(y_v, out_hbm.at[r])
                return 0

            lax.fori_loop(0, CCH, row_body, 0)

    return _slotfa


@jax.jit
def kernel(x, slot_assign, alphas, betas):
    b, c, h, w = x.shape
    s = alphas.shape[0]
    x2 = x.reshape(b * c, h * w)
    slot2 = slot_assign.reshape(b, h * w).astype(jnp.int32)
    a_t = alphas.T.reshape(-1)
    b_t = betas.T.reshape(-1)
    out2 = _build(b, c, h * w, s)(x2, slot2, a_t, b_t)
    return out2.reshape(x.shape)


# speedup vs baseline: 1.4270x; 1.4270x over previous
import functools

import jax
import jax.numpy as jnp
from jax import lax
from jax.experimental import pallas as pl
from jax.experimental.pallas import tpu as pltpu
from jax.experimental.pallas import tpu_sc as plsc

L = 16
NC, NS = 2, 16
NW = NC * NS


@functools.cache
def _build(B, C, HW, S):
    RPW = (B * C) // NW
    CCH = min(64, RPW)
    G = min(4, CCH)
    NG = CCH // G
    NCHUNK = RPW // CCH
    PV = HW // L
    mesh = plsc.VectorSubcoreMesh(core_axis_name="c", subcore_axis_name="s")

    @functools.partial(
        pl.kernel,
        mesh=mesh,
        out_type=jax.ShapeDtypeStruct((B * C, HW), jnp.float32),
        scratch_types=[
            pltpu.VMEM((HW,), jnp.int32),
            pltpu.VMEM((CCH * S,), jnp.float32),
            pltpu.VMEM((CCH * S,), jnp.float32),
            pltpu.VMEM((G, HW), jnp.float32),
            pltpu.VMEM((G, HW), jnp.float32),
            pltpu.VMEM((G, HW), jnp.float32),
            pltpu.VMEM((G, HW), jnp.float32),
            pltpu.SemaphoreType.DMA,
            pltpu.SemaphoreType.DMA,
            pltpu.SemaphoreType.DMA,
            pltpu.SemaphoreType.DMA,
        ],
        compiler_params=pltpu.CompilerParams(needs_layout_passes=False),
    )
    def _slotfa(x_hbm, slot_hbm, a_hbm, b_hbm, out_hbm,
                slot_v, at_v, bt_v, xb0, xb1, yb0, yb1,
                sx0, sx1, sy0, sy1):
        xb, yb, sx, sy = (xb0, xb1), (yb0, yb1), (sx0, sx1), (sy0, sy1)
        wid = lax.axis_index("s") * NC + lax.axis_index("c")
        r0 = wid * RPW
        bidx = r0 // C
        c0 = r0 % C
        pltpu.sync_copy(slot_hbm.at[bidx], slot_v)
        for cc in range(NCHUNK):
            cbase = c0 + cc * CCH
            rbase = r0 + cc * CCH
            pltpu.sync_copy(a_hbm.at[pl.ds(cbase * S, CCH * S)], at_v)
            pltpu.sync_copy(b_hbm.at[pl.ds(cbase * S, CCH * S)], bt_v)
            pltpu.async_copy(x_hbm.at[pl.ds(rbase, G)], xb0, sx0)
            pltpu.async_copy(x_hbm.at[pl.ds(rbase + G, G)], xb1, sx1)

            def gp_body(gp, _, rbase=rbase):
                for par in range(2):
                    g = gp * 2 + par
                    rg = rbase + g * G
                    pltpu.make_async_copy(
                        x_hbm.at[pl.ds(rg, G)], xb[par], sx[par]).wait()

                    @pl.when(gp >= 1)
                    def _():
                        pltpu.make_async_copy(
                            yb[par], out_hbm.at[pl.ds(rg, G)], sy[par]).wait()

                    base = g * G * S
                    cv = [jnp.full((L,), base + j * S, jnp.int32)
                          for j in range(G)]

                    def pv_body(pv, _):
                        o = pv * L
                        sl = slot_v[pl.ds(o, L)]
                        for j in range(G):
                            idx = sl + cv[j]
                            av = plsc.load_gather(at_v, [idx])
                            bv = plsc.load_gather(bt_v, [idx])
                            yb[par][j, pl.ds(o, L)] = (
                                av * xb[par][j, pl.ds(o, L)] + bv)
                        return 0

                    lax.fori_loop(0, PV, pv_body, 0, unroll=4)
                    pltpu.async_copy(yb[par], out_hbm.at[pl.ds(rg, G)],
                                     sy[par])

                    @pl.when(g + 2 < NG)
                    def _():
                        pltpu.async_copy(
                            x_hbm.at[pl.ds(rg + 2 * G, G)], xb[par], sx[par])

                return 0

            lax.fori_loop(0, NG // 2, gp_body, 0)
            pltpu.make_async_copy(
                yb0, out_hbm.at[pl.ds(rbase + (NG - 2) * G, G)], sy0).wait()
            pltpu.make_async_copy(
                yb1, out_hbm.at[pl.ds(rbase + (NG - 1) * G, G)], sy1).wait()

    return _slotfa


@jax.jit
def kernel(x, slot_assign, alphas, betas):
    b, c, h, w = x.shape
    s = alphas.shape[0]
    x2 = x.reshape(b * c, h * w)
    slot2 = slot_assign.reshape(b, h * w).astype(jnp.int32)
    a_t = alphas.T.reshape(-1)
    b_t = betas.T.reshape(-1)
    out2 = _build(b, c, h * w, s)(x2, slot2, a_t, b_t)
    return out2.reshape(x.shape)


# speedup vs baseline: 2.1164x; 1.4831x over previous
import functools

import jax
import jax.numpy as jnp
from jax import lax
from jax.experimental import pallas as pl
from jax.experimental.pallas import tpu as pltpu
from jax.experimental.pallas import tpu_sc as plsc

L = 16
NC, NS = 2, 16
NW = NC * NS


@functools.cache
def _build(B, C, HW, S):
    RPW = (B * C) // NW
    CCH = min(64, RPW)
    G = min(4, CCH)
    NG = CCH // G
    NCHUNK = RPW // CCH
    PV = HW // L
    mesh = plsc.VectorSubcoreMesh(core_axis_name="c", subcore_axis_name="s")

    @functools.partial(
        pl.kernel,
        mesh=mesh,
        out_type=jax.ShapeDtypeStruct((B * C, HW), jnp.float32),
        scratch_types=[
            pltpu.VMEM((HW,), jnp.int32),
            pltpu.VMEM((CCH * S,), jnp.float32),
            pltpu.VMEM((CCH * S,), jnp.float32),
            pltpu.VMEM((G, HW), jnp.float32),
            pltpu.VMEM((G, HW), jnp.float32),
            pltpu.VMEM((G, HW), jnp.float32),
            pltpu.VMEM((G, HW), jnp.float32),
            pltpu.SemaphoreType.DMA,
            pltpu.SemaphoreType.DMA,
            pltpu.SemaphoreType.DMA,
            pltpu.SemaphoreType.DMA,
        ],
        compiler_params=pltpu.CompilerParams(needs_layout_passes=False),
    )
    def _slotfa(x_hbm, slot_hbm, a_hbm, b_hbm, out_hbm,
                slot_v, at_v, bt_v, xb0, xb1, yb0, yb1,
                sx0, sx1, sy0, sy1):
        xb, yb, sx, sy = (xb0, xb1), (yb0, yb1), (sx0, sx1), (sy0, sy1)
        wid = lax.axis_index("s") * NC + lax.axis_index("c")
        r0 = wid * RPW
        bidx = r0 // C
        c0 = r0 % C
        pltpu.sync_copy(slot_hbm.at[bidx], slot_v)
        for cc in range(NCHUNK):
            cbase = c0 + cc * CCH
            rbase = r0 + cc * CCH
            pltpu.sync_copy(a_hbm.at[pl.ds(cbase * S, CCH * S)], at_v)
            pltpu.sync_copy(b_hbm.at[pl.ds(cbase * S, CCH * S)], bt_v)
            pltpu.async_copy(x_hbm.at[pl.ds(rbase, G)], xb0, sx0)
            pltpu.async_copy(x_hbm.at[pl.ds(rbase + G, G)], xb1, sx1)

            def gp_body(gp, _, rbase=rbase):
                for par in range(2):
                    g = gp * 2 + par
                    rg = rbase + g * G
                    pltpu.make_async_copy(
                        x_hbm.at[pl.ds(rg, G)], xb[par], sx[par]).wait()

                    @pl.when(gp >= 1)
                    def _():
                        pltpu.make_async_copy(
                            yb[par], out_hbm.at[pl.ds(rg, G)], sy[par]).wait()

                    base = g * G * S
                    cv = [jnp.full((L,), base + j * S, jnp.int32)
                          for j in range(G)]

                    @plsc.parallel_loop(0, PV, 1, unroll=8)
                    def pv_body(pv):
                        o = pv * L
                        sl = slot_v[pl.ds(o, L)]
                        for j in range(G):
                            idx = sl + cv[j]
                            av = plsc.load_gather(at_v, [idx])
                            bv = plsc.load_gather(bt_v, [idx])
                            yb[par][j, pl.ds(o, L)] = (
                                av * xb[par][j, pl.ds(o, L)] + bv)
                    pltpu.async_copy(yb[par], out_hbm.at[pl.ds(rg, G)],
                                     sy[par])

                    @pl.when(g + 2 < NG)
                    def _():
                        pltpu.async_copy(
                            x_hbm.at[pl.ds(rg + 2 * G, G)], xb[par], sx[par])

                return 0

            lax.fori_loop(0, NG // 2, gp_body, 0)
            pltpu.make_async_copy(
                yb0, out_hbm.at[pl.ds(rbase + (NG - 2) * G, G)], sy0).wait()
            pltpu.make_async_copy(
                yb1, out_hbm.at[pl.ds(rbase + (NG - 1) * G, G)], sy1).wait()

    return _slotfa


@jax.jit
def kernel(x, slot_assign, alphas, betas):
    b, c, h, w = x.shape
    s = alphas.shape[0]
    x2 = x.reshape(b * c, h * w)
    slot2 = slot_assign.reshape(b, h * w).astype(jnp.int32)
    a_t = alphas.T.reshape(-1)
    b_t = betas.T.reshape(-1)
    out2 = _build(b, c, h * w, s)(x2, slot2, a_t, b_t)
    return out2.reshape(x.shape)
